# baseline (speedup 1.0000x reference)
import functools

import jax
import jax.numpy as jnp
from jax import lax
from jax.experimental import pallas as pl
from jax.experimental.pallas import tpu as pltpu
from jax.experimental.pallas import tpu_sc as plsc


def _segsum_sc(flat, B, P, D):
    info = plsc.get_sparse_core_info()
    NC, NS, L = info.num_cores, info.num_subcores, info.num_lanes
    NW = NC * NS
    BPW = B // NW
    CB = 128 // P
    RPC = CB * P
    NCHUNK = BPW // CB
    mesh = plsc.VectorSubcoreMesh(core_axis_name="c", subcore_axis_name="s")

    @functools.partial(
        pl.kernel,
        out_type=jax.ShapeDtypeStruct((B, D), jnp.float32),
        mesh=mesh,
        scratch_types=[
            pltpu.VMEM((NCHUNK, RPC), jnp.int32),
            pltpu.VMEM((RPC, D), jnp.float32),
            pltpu.VMEM_SHARED((NS * BPW, D), jnp.float32),
        ],
    )
    def seg(flat_hbm, out_hbm, idx_v, buf, acc):
        cid = lax.axis_index("c")
        sid = lax.axis_index("s")
        wid = cid * NS + sid
        in_base = wid * (BPW * P)
        acc_base = sid * BPW

        def zrow(r, _):
            for j in range(D // L):
                buf[r, pl.ds(j * L, L)] = jnp.zeros((L,), jnp.float32)
            return _
        lax.fori_loop(0, RPC, zrow, 0)
        pltpu.sync_copy(buf, acc.at[pl.ds(acc_base, BPW)])

        def irow(c, _):
            for j in range(RPC // L):
                val = acc_base + c * CB + j // (P // L)
                idx_v[c, pl.ds(j * L, L)] = jnp.zeros((L,), jnp.int32) + val
            return _
        lax.fori_loop(0, NCHUNK, irow, 0)

        def chunk(c, _):
            pltpu.sync_copy(flat_hbm.at[pl.ds(in_base + c * RPC, RPC)], buf)
            pltpu.sync_copy(buf, acc.at[idx_v.at[c]], add=True)
            return _
        lax.fori_loop(0, NCHUNK, chunk, 0)

        pltpu.sync_copy(acc.at[pl.ds(acc_base, BPW)],
                        out_hbm.at[pl.ds(wid * BPW, BPW)])

    return seg(flat)


def _mlp_body(s_ref, w_ref, b_ref, o_ref):
    y = jax.lax.dot_general(
        s_ref[...], w_ref[...], (((1,), (1,)), ((), ())),
        preferred_element_type=jnp.float32,
        precision=jax.lax.Precision.HIGHEST,
    )
    o_ref[...] = y + b_ref[...]


def kernel(pert_batch, W, b):
    B, P, D = pert_batch.shape
    OUT = W.shape[0]
    flat = pert_batch.reshape(B * P, D)
    s = _segsum_sc(flat, B, P, D)
    bias = (P * b).reshape(1, OUT)
    return pl.pallas_call(
        _mlp_body,
        in_specs=[
            pl.BlockSpec((B, D), lambda: (0, 0)),
            pl.BlockSpec((OUT, D), lambda: (0, 0)),
            pl.BlockSpec((1, OUT), lambda: (0, 0)),
        ],
        out_specs=pl.BlockSpec((B, OUT), lambda: (0, 0)),
        out_shape=jax.ShapeDtypeStruct((B, OUT), jnp.float32),
    )(s, W, bias)


# speedup vs baseline: 1.2392x; 1.2392x over previous
import functools

import jax
import jax.numpy as jnp
from jax import lax
from jax.experimental import pallas as pl
from jax.experimental.pallas import tpu as pltpu
from jax.experimental.pallas import tpu_sc as plsc


def _segsum_sc(flat, B, P, D):
    info = plsc.get_sparse_core_info()
    NC, NS, L = info.num_cores, info.num_subcores, info.num_lanes
    NW = NC * NS
    BPW = B // NW
    RPC = 256
    KI = RPC // 128
    CB = RPC // P
    NCHUNK = (BPW * P) // RPC
    mesh = plsc.VectorSubcoreMesh(core_axis_name="c", subcore_axis_name="s")

    @functools.partial(
        pl.kernel,
        out_type=jax.ShapeDtypeStruct((B, D), jnp.float32),
        mesh=mesh,
        scratch_types=[
            pltpu.VMEM((NCHUNK, KI, 128), jnp.int32),
            pltpu.VMEM((RPC, D), jnp.float32),
            pltpu.VMEM((RPC, D), jnp.float32),
            pltpu.VMEM_SHARED((NS * BPW, D), jnp.float32),
            pltpu.SemaphoreType.DMA,
            pltpu.SemaphoreType.DMA,
            pltpu.SemaphoreType.DMA,
        ],
    )
    def seg(flat_hbm, out_hbm, idx_v, buf0, buf1, acc, sem0, sem1, sem2):
        sid = lax.axis_index("s")
        wid = lax.axis_index("c") * NS + sid
        in_base = wid * (BPW * P)
        acc_base = sid * BPW

        def zrow(r, _):
            for j in range(D // L):
                buf0[r, pl.ds(j * L, L)] = jnp.zeros((L,), jnp.float32)
            return _
        lax.fori_loop(0, BPW, zrow, 0)
        pltpu.sync_copy(buf0.at[pl.ds(0, BPW)], acc.at[pl.ds(acc_base, BPW)])

        def irow(c, _):
            for k in range(KI):
                for j in range(128 // L):
                    val = acc_base + c * CB + k * (128 // P) + j // (P // L)
                    idx_v[c, k, pl.ds(j * L, L)] = jnp.zeros((L,), jnp.int32) + val
            return _
        lax.fori_loop(0, NCHUNK, irow, 0)

        bufs = (buf0, buf1)
        sems = (sem0, sem1)
        d_cur = pltpu.async_copy(
            flat_hbm.at[pl.ds(in_base, RPC)], buf0, sem0)
        for g in range(NCHUNK):
            cur = bufs[g % 2]
            d_cur.wait()
            if g + 1 < NCHUNK:
                d_next = pltpu.async_copy(
                    flat_hbm.at[pl.ds(in_base + (g + 1) * RPC, RPC)],
                    bufs[(g + 1) % 2], sems[(g + 1) % 2])
            scat = [
                pltpu.async_copy(cur.at[pl.ds(k * 128, 128)],
                                 acc.at[idx_v.at[g, k]], sem2, add=True)
                for k in range(KI)
            ]
            for dsc in scat:
                dsc.wait()
            if g + 1 < NCHUNK:
                d_cur = d_next

        pltpu.sync_copy(acc.at[pl.ds(acc_base, BPW)],
                        out_hbm.at[pl.ds(wid * BPW, BPW)])

    return seg(flat)


def _mlp_body(s_ref, w_ref, b_ref, o_ref):
    y = jax.lax.dot_general(
        s_ref[...], w_ref[...], (((1,), (1,)), ((), ())),
        preferred_element_type=jnp.float32,
        precision=jax.lax.Precision.HIGHEST,
    )
    o_ref[...] = y + b_ref[...]


def kernel(pert_batch, W, b):
    B, P, D = pert_batch.shape
    OUT = W.shape[0]
    flat = pert_batch.reshape(B * P, D)
    s = _segsum_sc(flat, B, P, D)
    bias = (P * b).reshape(1, OUT)
    return pl.pallas_call(
        _mlp_body,
        in_specs=[
            pl.BlockSpec((B, D), lambda: (0, 0)),
            pl.BlockSpec((OUT, D), lambda: (0, 0)),
            pl.BlockSpec((1, OUT), lambda: (0, 0)),
        ],
        out_specs=pl.BlockSpec((B, OUT), lambda: (0, 0)),
        out_shape=jax.ShapeDtypeStruct((B, OUT), jnp.float32),
    )(s, W, bias)


# speedup vs baseline: 1.3196x; 1.0649x over previous
import functools

import jax
import jax.numpy as jnp
from jax import lax
from jax.experimental import pallas as pl
from jax.experimental.pallas import tpu as pltpu
from jax.experimental.pallas import tpu_sc as plsc


def _segsum_sc(flat, B, P, D):
    info = plsc.get_sparse_core_info()
    NC, NS, L = info.num_cores, info.num_subcores, info.num_lanes
    NW = NC * NS
    BPW = B // NW
    RPC = 256
    KI = RPC // 128
    CB = RPC // P
    NCHUNK = (BPW * P) // RPC
    NBUF = 3
    mesh = plsc.VectorSubcoreMesh(core_axis_name="c", subcore_axis_name="s")

    @functools.partial(
        pl.kernel,
        out_type=jax.ShapeDtypeStruct((B, D), jnp.float32),
        mesh=mesh,
        scratch_types=[
            pltpu.VMEM((NCHUNK, KI, 128), jnp.int32),
            [pltpu.VMEM((RPC, D), jnp.float32)] * NBUF,
            pltpu.VMEM_SHARED((NS * BPW, D), jnp.float32),
            [pltpu.SemaphoreType.DMA] * NBUF,
            [pltpu.SemaphoreType.DMA] * NBUF,
        ],
    )
    def seg(flat_hbm, out_hbm, idx_v, bufs, acc, hsems, ssems):
        sid = lax.axis_index("s")
        wid = lax.axis_index("c") * NS + sid
        in_base = wid * (BPW * P)
        acc_base = sid * BPW

        def hbm_start(g, ):
            return pltpu.async_copy(
                flat_hbm.at[pl.ds(in_base + g * RPC, RPC)],
                bufs[g % NBUF], hsems[g % NBUF])

        def scat_start(g):
            return [
                pltpu.async_copy(bufs[g % NBUF].at[pl.ds(k * 128, 128)],
                                 acc.at[idx_v.at[g, k]],
                                 ssems[g % NBUF], add=True)
                for k in range(KI)
            ]

        d0 = hbm_start(0)

        zbuf = bufs[NBUF - 1]
        def zrow(r, _):
            for j in range(D // L):
                zbuf[r, pl.ds(j * L, L)] = jnp.zeros((L,), jnp.float32)
            return _
        lax.fori_loop(0, BPW, zrow, 0)
        pltpu.sync_copy(zbuf.at[pl.ds(0, BPW)], acc.at[pl.ds(acc_base, BPW)])

        def irow(c, _):
            for k in range(KI):
                for j in range(128 // L):
                    val = acc_base + c * CB + k * (128 // P) + j // (P // L)
                    idx_v[c, k, pl.ds(j * L, L)] = jnp.zeros((L,), jnp.int32) + val
            return _
        lax.fori_loop(0, NCHUNK, irow, 0)

        d1 = hbm_start(1)

        hbm_d = [d0, d1]
        prev_scat = None
        for g in range(NCHUNK):
            hbm_d.pop(0).wait()
            cur_scat = scat_start(g)
            if prev_scat is not None:
                for dsc in prev_scat:
                    dsc.wait()
            if g + 2 < NCHUNK:
                hbm_d.append(hbm_start(g + 2))
            prev_scat = cur_scat
        for dsc in prev_scat:
            dsc.wait()

        pltpu.sync_copy(acc.at[pl.ds(acc_base, BPW)],
                        out_hbm.at[pl.ds(wid * BPW, BPW)])

    return seg(flat)


def _mlp_body(s_ref, w_ref, b_ref, o_ref):
    y = jax.lax.dot_general(
        s_ref[...], w_ref[...], (((1,), (1,)), ((), ())),
        preferred_element_type=jnp.float32,
        precision=jax.lax.Precision.HIGHEST,
    )
    o_ref[...] = y + b_ref[...]


def kernel(pert_batch, W, b):
    B, P, D = pert_batch.shape
    OUT = W.shape[0]
    flat = pert_batch.reshape(B * P, D)
    s = _segsum_sc(flat, B, P, D)
    bias = (P * b).reshape(1, OUT)
    return pl.pallas_call(
        _mlp_body,
        in_specs=[
            pl.BlockSpec((B, D), lambda: (0, 0)),
            pl.BlockSpec((OUT, D), lambda: (0, 0)),
            pl.BlockSpec((1, OUT), lambda: (0, 0)),
        ],
        out_specs=pl.BlockSpec((B, OUT), lambda: (0, 0)),
        out_shape=jax.ShapeDtypeStruct((B, OUT), jnp.float32),
    )(s, W, bias)


# speedup vs baseline: 1.8368x; 1.3919x over previous
import functools

import jax
import jax.numpy as jnp
from jax import lax
from jax.experimental import pallas as pl
from jax.experimental.pallas import tpu as pltpu
from jax.experimental.pallas import tpu_sc as plsc


def _segsum_sc(flat, B, P, D):
    info = plsc.get_sparse_core_info()
    NC, NS, L = info.num_cores, info.num_subcores, info.num_lanes
    NW = NC * NS
    NV = D // L
    BPW = B // NW
    RPC = 256
    SPC = RPC // P
    NCHUNK = (BPW * P) // RPC
    NBUF = 3
    mesh = plsc.VectorSubcoreMesh(core_axis_name="c", subcore_axis_name="s")

    @functools.partial(
        pl.kernel,
        out_type=jax.ShapeDtypeStruct((B, D), jnp.float32),
        mesh=mesh,
        scratch_types=[
            [pltpu.VMEM((RPC, D), jnp.float32)] * NBUF,
            pltpu.VMEM((BPW, D), jnp.float32),
            [pltpu.SemaphoreType.DMA] * NBUF,
        ],
    )
    def seg(flat_hbm, out_hbm, bufs, res, hsems):
        sid = lax.axis_index("s")
        wid = lax.axis_index("c") * NS + sid
        in_base = wid * (BPW * P)

        def hbm_start(g):
            return pltpu.async_copy(
                flat_hbm.at[pl.ds(in_base + g * RPC, RPC)],
                bufs[g % NBUF], hsems[g % NBUF])

        def reduce_chunk(buf, g):
            def seg_body(t, _):
                base = t * P
                acc = [buf[base, pl.ds(j * L, L)] for j in range(NV)]
                def row_body(r, acc):
                    return tuple(
                        acc[j] + buf[base + r, pl.ds(j * L, L)]
                        for j in range(NV)
                    )
                acc = lax.fori_loop(1, P, row_body, tuple(acc))
                for j in range(NV):
                    res[g * SPC + t, pl.ds(j * L, L)] = acc[j]
                return _
            lax.fori_loop(0, SPC, seg_body, 0)

        hbm_d = [hbm_start(0), hbm_start(1)]
        for gg in range(0, NCHUNK, NBUF):
            for b in range(NBUF):
                g = gg + b
                if g >= NCHUNK:
                    break
                hbm_d.pop(0).wait()
                if g + 2 < NCHUNK:
                    hbm_d.append(hbm_start(g + 2))
                reduce_chunk(bufs[g % NBUF], g)

        pltpu.sync_copy(res, out_hbm.at[pl.ds(wid * BPW, BPW)])

    return seg(flat)


def _mlp_body(s_ref, w_ref, b_ref, o_ref):
    y = jax.lax.dot_general(
        s_ref[...], w_ref[...], (((1,), (1,)), ((), ())),
        preferred_element_type=jnp.float32,
        precision=jax.lax.Precision.HIGHEST,
    )
    o_ref[...] = y + b_ref[...]


def kernel(pert_batch, W, b):
    B, P, D = pert_batch.shape
    OUT = W.shape[0]
    flat = pert_batch.reshape(B * P, D)
    s = _segsum_sc(flat, B, P, D)
    bias = (P * b).reshape(1, OUT)
    return pl.pallas_call(
        _mlp_body,
        in_specs=[
            pl.BlockSpec((B, D), lambda: (0, 0)),
            pl.BlockSpec((OUT, D), lambda: (0, 0)),
            pl.BlockSpec((1, OUT), lambda: (0, 0)),
        ],
        out_specs=pl.BlockSpec((B, OUT), lambda: (0, 0)),
        out_shape=jax.ShapeDtypeStruct((B, OUT), jnp.float32),
    )(s, W, bias)


# speedup vs baseline: 2.0242x; 1.1021x over previous
import functools

import jax
import jax.numpy as jnp
from jax import lax
from jax.experimental import pallas as pl
from jax.experimental.pallas import tpu as pltpu
from jax.experimental.pallas import tpu_sc as plsc


def _segsum_sc(flat, BS, P, D):
    info = plsc.get_sparse_core_info()
    NC, NS, L = info.num_cores, info.num_subcores, info.num_lanes
    NW = NC * NS
    NV = D // L
    BPW = BS // NW
    RPC = 256
    SPC = RPC // P
    NCHUNK = (BPW * P) // RPC
    NBUF = 3
    mesh = plsc.VectorSubcoreMesh(core_axis_name="c", subcore_axis_name="s")

    @functools.partial(
        pl.kernel,
        out_type=jax.ShapeDtypeStruct((BS, D), jnp.float32),
        mesh=mesh,
        scratch_types=[
            [pltpu.VMEM((RPC, D), jnp.float32)] * NBUF,
            pltpu.VMEM((BPW, D), jnp.float32),
            [pltpu.SemaphoreType.DMA] * NBUF,
        ],
    )
    def seg(flat_hbm, out_hbm, bufs, res, hsems):
        sid = lax.axis_index("s")
        wid = lax.axis_index("c") * NS + sid
        in_base = wid * (BPW * P)

        def hbm_start(g):
            return pltpu.async_copy(
                flat_hbm.at[pl.ds(in_base + g * RPC, RPC)],
                bufs[g % NBUF], hsems[g % NBUF])

        def reduce_chunk(buf, g):
            def seg_body(t, _):
                base = t * P
                acc = [buf[base, pl.ds(j * L, L)] for j in range(NV)]
                def row_body(r, acc):
                    return tuple(
                        acc[j] + buf[base + r, pl.ds(j * L, L)]
                        for j in range(NV)
                    )
                acc = lax.fori_loop(1, P, row_body, tuple(acc))
                for j in range(NV):
                    res[g * SPC + t, pl.ds(j * L, L)] = acc[j]
                return _
            lax.fori_loop(0, SPC, seg_body, 0)

        hbm_d = [hbm_start(0), hbm_start(1)]
        for gg in range(0, NCHUNK, NBUF):
            for b in range(NBUF):
                g = gg + b
                if g >= NCHUNK:
                    break
                hbm_d.pop(0).wait()
                if g + 2 < NCHUNK:
                    hbm_d.append(hbm_start(g + 2))
                reduce_chunk(bufs[g % NBUF], g)

        pltpu.sync_copy(res, out_hbm.at[pl.ds(wid * BPW, BPW)])

    return seg(flat)


def _linear(s, w, bias):
    y = jax.lax.dot_general(
        s, w, (((1,), (1,)), ((), ())),
        preferred_element_type=jnp.float32,
        precision=jax.lax.Precision.HIGHEST,
    )
    return y + bias


def _mlp_body(s_ref, w_ref, b_ref, o_ref):
    o_ref[...] = _linear(s_ref[...], w_ref[...], b_ref[...])


def _fused_body(x_ref, w_ref, b_ref, o_ref):
    s = jnp.sum(x_ref[...], axis=1)
    o_ref[...] = _linear(s, w_ref[...], b_ref[...])


def kernel(pert_batch, W, b):
    B, P, D = pert_batch.shape
    OUT = W.shape[0]
    BS = 1024
    BB = 512
    bias = (P * b).reshape(1, OUT)
    flat = pert_batch.reshape(B * P, D)

    s_sc = _segsum_sc(flat, BS, P, D)

    nblk = (B - BS) // BB
    y_tc = pl.pallas_call(
        _fused_body,
        grid=(nblk,),
        in_specs=[
            pl.BlockSpec((BB, P, D), lambda i, o=BS // BB: (o + i, 0, 0)),
            pl.BlockSpec((OUT, D), lambda i: (0, 0)),
            pl.BlockSpec((1, OUT), lambda i: (0, 0)),
        ],
        out_specs=pl.BlockSpec((BB, OUT), lambda i: (i, 0)),
        out_shape=jax.ShapeDtypeStruct((B - BS, OUT), jnp.float32),
    )(pert_batch, W, bias)

    y_sc = pl.pallas_call(
        _mlp_body,
        in_specs=[
            pl.BlockSpec((BS, D), lambda: (0, 0)),
            pl.BlockSpec((OUT, D), lambda: (0, 0)),
            pl.BlockSpec((1, OUT), lambda: (0, 0)),
        ],
        out_specs=pl.BlockSpec((BS, OUT), lambda: (0, 0)),
        out_shape=jax.ShapeDtypeStruct((BS, OUT), jnp.float32),
    )(s_sc, W, bias)

    return jnp.concatenate([y_sc, y_tc], axis=0)
